# baseline (speedup 1.0000x reference)
import jax
import jax.numpy as jnp
from jax.experimental import pallas as pl
from jax.experimental.pallas import tpu as pltpu

_NQ = 4
_K = 1024
_D = 256
_BT = 256


def _rvq_block(x_ref, emb_ref, q_ref, idx_ref, loss_ref, e2_ref):
    i = pl.program_id(0)

    @pl.when(i == 0)
    def _init():
        emb = emb_ref[...]
        e2_ref[...] = jnp.sum(emb * emb, axis=2)
        loss_ref[...] = jnp.zeros_like(loss_ref)

    r = x_ref[...]
    acc = jnp.zeros_like(r)
    s = jnp.float32(0.0)
    for st in range(_NQ):
        e = emb_ref[st]
        x2 = jnp.sum(r * r, axis=1, keepdims=True)
        mm = jax.lax.dot_general(
            r, e, (((1,), (1,)), ((), ())),
            preferred_element_type=jnp.float32,
            precision=jax.lax.Precision.DEFAULT)
        dist = (x2 + e2_ref[st][None, :]) - 2.0 * mm
        idx = jnp.argmin(dist, axis=1).astype(jnp.int32)
        idx_ref[st] = idx
        oh = (jax.lax.broadcasted_iota(jnp.int32, (_BT, _K), 1)
              == idx[:, None]).astype(jnp.float32)
        q = jax.lax.dot_general(
            oh, e, (((1,), (0,)), ((), ())),
            preferred_element_type=jnp.float32,
            precision=jax.lax.Precision.HIGHEST)
        s = s + jnp.sum((q - r) ** 2)
        qst = r + (q - r)
        acc = acc + qst
        r = r - qst
    q_ref[...] = acc
    loss_ref[...] += s


def kernel(x, embeddings):
    shape = x.shape
    flat = x.reshape(-1, _D)
    nt = flat.shape[0]
    q, idx, loss = pl.pallas_call(
        _rvq_block,
        grid=(nt // _BT,),
        in_specs=[
            pl.BlockSpec((_BT, _D), lambda i: (i, 0)),
            pl.BlockSpec((_NQ, _K, _D), lambda i: (0, 0, 0)),
        ],
        out_specs=[
            pl.BlockSpec((_BT, _D), lambda i: (i, 0)),
            pl.BlockSpec((_NQ, _BT), lambda i: (0, i)),
            pl.BlockSpec((8, 128), lambda i: (0, 0)),
        ],
        out_shape=[
            jax.ShapeDtypeStruct((nt, _D), jnp.float32),
            jax.ShapeDtypeStruct((_NQ, nt), jnp.int32),
            jax.ShapeDtypeStruct((8, 128), jnp.float32),
        ],
        scratch_shapes=[pltpu.VMEM((_NQ, _K), jnp.float32)],
    )(flat, embeddings)
    mean_sq = loss[0, 0] / jnp.float32(nt * _D)
    vq_loss = 1.25 * mean_sq
    commit_loss = 0.25 * mean_sq
    cb_loss = mean_sq
    indices = idx.reshape((_NQ,) + shape[:-1])
    return (q.reshape(shape), indices, vq_loss, commit_loss, cb_loss)


# speedup vs baseline: 2.4789x; 2.4789x over previous
import jax
import jax.numpy as jnp
from jax.experimental import pallas as pl
from jax.experimental.pallas import tpu as pltpu

_NQ = 4
_K = 1024
_D = 256
_BT = 1024
_ST = 256


def _prep_block(emb_ref, e2_ref, es_ref):
    emb = emb_ref[...]
    e2_ref[...] = jnp.sum(emb * emb, axis=2)
    s0 = emb.astype(jnp.bfloat16)
    r1 = emb - s0.astype(jnp.float32)
    s1 = r1.astype(jnp.bfloat16)
    r2 = r1 - s1.astype(jnp.float32)
    es_ref[0] = s0
    es_ref[1] = s1
    es_ref[2] = r2.astype(jnp.bfloat16)


def _rvq_block(x_ref, emb_ref, e2_ref, es_ref, q_ref, idx_ref, loss_ref):
    nsub = _BT // _ST
    rs = [x_ref[pl.ds(t * _ST, _ST), :] for t in range(nsub)]
    accs = [jnp.zeros_like(rs[t]) for t in range(nsub)]
    s = jnp.float32(0.0)
    for st in range(_NQ):
        for t in range(nsub):
            r = rs[t]
            e = emb_ref[st]
            x2 = jnp.sum(r * r, axis=1, keepdims=True)
            mm = jax.lax.dot_general(
                r, e, (((1,), (1,)), ((), ())),
                preferred_element_type=jnp.float32,
                precision=jax.lax.Precision.DEFAULT)
            dist = (x2 + e2_ref[st][None, :]) - 2.0 * mm
            idx = jnp.argmin(dist, axis=1).astype(jnp.int32)
            idx_ref[st, pl.ds(t * _ST, _ST)] = idx
            oh = (jax.lax.broadcasted_iota(jnp.int32, (_ST, _K), 1)
                  == idx[:, None]).astype(jnp.bfloat16)
            dims = (((1,), (0,)), ((), ()))
            q0 = jax.lax.dot_general(oh, es_ref[0, st], dims,
                                     preferred_element_type=jnp.float32)
            q1 = jax.lax.dot_general(oh, es_ref[1, st], dims,
                                     preferred_element_type=jnp.float32)
            q2 = jax.lax.dot_general(oh, es_ref[2, st], dims,
                                     preferred_element_type=jnp.float32)
            q = (q0 + q1) + q2
            s = s + jnp.sum((q - r) ** 2)
            qst = r + (q - r)
            accs[t] = accs[t] + qst
            rs[t] = r - qst
    for t in range(nsub):
        q_ref[pl.ds(t * _ST, _ST), :] = accs[t]
    loss_ref[...] = jnp.full((1, 1, 128), s, jnp.float32)


def kernel(x, embeddings):
    shape = x.shape
    flat = x.reshape(-1, _D)
    nt = flat.shape[0]
    nblk = nt // _BT
    e2, es = pl.pallas_call(
        _prep_block,
        in_specs=[pl.BlockSpec((_NQ, _K, _D), lambda: (0, 0, 0))],
        out_specs=[pl.BlockSpec((_NQ, _K), lambda: (0, 0)),
                   pl.BlockSpec((3, _NQ, _K, _D), lambda: (0, 0, 0, 0))],
        out_shape=[jax.ShapeDtypeStruct((_NQ, _K), jnp.float32),
                   jax.ShapeDtypeStruct((3, _NQ, _K, _D), jnp.bfloat16)],
    )(embeddings)
    q, idx, loss = pl.pallas_call(
        _rvq_block,
        grid=(nblk,),
        in_specs=[
            pl.BlockSpec((_BT, _D), lambda i: (i, 0)),
            pl.BlockSpec((_NQ, _K, _D), lambda i: (0, 0, 0)),
            pl.BlockSpec((_NQ, _K), lambda i: (0, 0)),
            pl.BlockSpec((3, _NQ, _K, _D), lambda i: (0, 0, 0, 0)),
        ],
        out_specs=[
            pl.BlockSpec((_BT, _D), lambda i: (i, 0)),
            pl.BlockSpec((_NQ, _BT), lambda i: (0, i)),
            pl.BlockSpec((1, 1, 128), lambda i: (i, 0, 0)),
        ],
        out_shape=[
            jax.ShapeDtypeStruct((nt, _D), jnp.float32),
            jax.ShapeDtypeStruct((_NQ, nt), jnp.int32),
            jax.ShapeDtypeStruct((nblk, 1, 128), jnp.float32),
        ],
        compiler_params=pltpu.CompilerParams(
            dimension_semantics=("parallel",)),
    )(flat, embeddings, e2, es)
    mean_sq = jnp.sum(loss[:, 0, 0]) / jnp.float32(nt * _D)
    vq_loss = 1.25 * mean_sq
    commit_loss = 0.25 * mean_sq
    cb_loss = mean_sq
    indices = idx.reshape((_NQ,) + shape[:-1])
    return (q.reshape(shape), indices, vq_loss, commit_loss, cb_loss)
